# Initial kernel scaffold; baseline (speedup 1.0000x reference)
import jax
import jax.numpy as jnp
from jax.experimental import pallas as pl


def kernel(pos):
    raise NotImplementedError("write your pallas kernel here")


# speedup vs baseline: 1.6352x; 1.6352x over previous
import functools

import numpy as np

import jax
import jax.numpy as jnp
from jax.experimental import pallas as pl
from jax.experimental.pallas import tpu as pltpu

_LANES = 128
_E = 32
_LBASE = 1.25
_P = 8
_KB = 1024

_RN = 12582912.0
_C0 = 6.283054
_C1 = -41.33112
_C2 = 81.36547
_C3 = -74.47079
_C4 = 32.768528


@functools.lru_cache(maxsize=None)
def _tables():
    c = np.arange(_P * _LANES)
    e = c % _E
    i = e // 2
    fsc = ((np.float64(_LBASE) ** i) / 2.0).astype(np.float32)
    psc = np.where(e % 2 == 1, 0.25, 0.0).astype(np.float32)
    p = c // _LANES
    l = c % _LANES
    spread = (np.arange(_E)[:, None] == (4 * p + l // _E)[None, :])
    return (jnp.asarray(fsc.reshape(1, _P * _LANES)),
            jnp.asarray(psc.reshape(1, _P * _LANES)),
            jnp.asarray(spread.astype(np.float32)))


def _pe_kernel(pos_ref, spread_ref, fsc_ref, psc_ref, out_ref):
    s = jnp.dot(pos_ref[...], spread_ref[...],
                preferred_element_type=jnp.float32)
    a = s * fsc_ref[...] + psc_ref[...]
    k = (a + _RN) - _RN
    f = a - k
    z = f * f
    p = _C4
    p = p * z + _C3
    p = p * z + _C2
    p = p * z + _C1
    p = p * z + _C0
    r = p * f
    for j in range(_P):
        out_ref[:, j, :] = r[:, j * _LANES:(j + 1) * _LANES]


def _round_up(x, m):
    return (x + m - 1) // m * m


def kernel(pos):
    pos = jnp.asarray(pos, dtype=jnp.float32)
    B = pos.shape[0]

    tb = _KB * _E
    b_pad = _round_up(B, tb)
    grid = b_pad // tb
    pos_p = pos if b_pad == B else jnp.pad(pos, (0, b_pad - B))
    pos32 = pos_p.reshape(b_pad // _E, _E)

    fsc, psc, spread = _tables()

    out3 = pl.pallas_call(
        _pe_kernel,
        out_shape=jax.ShapeDtypeStruct((b_pad // _E, _P, _LANES), jnp.float32),
        grid=(grid,),
        in_specs=[
            pl.BlockSpec((_KB, _E), lambda i: (i, 0)),
            pl.BlockSpec((_E, _P * _LANES), lambda i: (0, 0)),
            pl.BlockSpec((1, _P * _LANES), lambda i: (0, 0)),
            pl.BlockSpec((1, _P * _LANES), lambda i: (0, 0)),
        ],
        out_specs=pl.BlockSpec((_KB, _P, _LANES), lambda i: (i, 0, 0)),
        compiler_params=pltpu.CompilerParams(
            dimension_semantics=("parallel",),
        ),
    )(pos32, spread, fsc, psc)

    out = out3.reshape(b_pad, _E)
    return out if b_pad == B else out[:B]


# speedup vs baseline: 14.9361x; 9.1340x over previous
import functools

import numpy as np

import jax
import jax.numpy as jnp
from jax.experimental import pallas as pl
from jax.experimental.pallas import tpu as pltpu

_E = 32
_LBASE = 1.25
_W = 4096
_S = 8
_TB = _S * _W

_RN = 12582912.0
_C0 = 6.283054
_C1 = -41.33112
_C2 = 81.36547
_C3 = -74.47079
_C4 = 32.768528


@functools.lru_cache(maxsize=None)
def _tables():
    j = np.arange(_E)
    fsc = ((np.float64(_LBASE) ** (j // 2)) / 2.0).astype(np.float32)
    psc = np.where(j % 2 == 1, 0.25, 0.0).astype(np.float32)
    fsc_full = np.broadcast_to(fsc[:, None], (_E, _W)).copy()
    psc_full = np.broadcast_to(psc[:, None], (_E, _W)).copy()
    return jnp.asarray(fsc_full), jnp.asarray(psc_full)


def _pe_kernel(pos_ref, fsc_ref, psc_ref, out_ref):
    fsc = fsc_ref[...]
    psc = psc_ref[...]
    for s in range(_S):
        pv = pos_ref[s:s + 1, :]
        a = pv * fsc + psc
        k = (a + _RN) - _RN
        f = a - k
        z = f * f
        p = _C4
        p = p * z + _C3
        p = p * z + _C2
        p = p * z + _C1
        p = p * z + _C0
        out_ref[:, pl.ds(s * _W, _W)] = p * f


def _round_up(x, m):
    return (x + m - 1) // m * m


def kernel(pos):
    pos = jnp.asarray(pos, dtype=jnp.float32)
    B = pos.shape[0]

    b_pad = _round_up(B, _TB)
    grid = b_pad // _TB
    pos_p = pos if b_pad == B else jnp.pad(pos, (0, b_pad - B))
    pos2 = pos_p.reshape(b_pad // _W, _W)

    fsc, psc = _tables()

    out_t = pl.pallas_call(
        _pe_kernel,
        out_shape=jax.ShapeDtypeStruct((_E, b_pad), jnp.float32),
        grid=(grid,),
        in_specs=[
            pl.BlockSpec((_S, _W), lambda i: (i, 0)),
            pl.BlockSpec((_E, _W), lambda i: (0, 0)),
            pl.BlockSpec((_E, _W), lambda i: (0, 0)),
        ],
        out_specs=pl.BlockSpec((_E, _TB), lambda i: (0, i)),
        compiler_params=pltpu.CompilerParams(
            dimension_semantics=("parallel",),
        ),
    )(pos2, fsc, psc)

    out = out_t.T
    return out if b_pad == B else out[:B]
